# baseline (speedup 1.0000x reference)
import jax
import jax.numpy as jnp
from jax import lax
from jax.experimental import pallas as pl
from jax.experimental.pallas import tpu as pltpu
from jax.experimental.pallas import tpu_sc as plsc

B = 16384
F = 32
H = 64
LAMBDA = 0.001

NU = 359347
NI = 292589

_NC = 2
_NS = 16
_NW = _NC * _NS
_CHUNK = 128
_BPW = B // _NW
_CPW = _BPW // _CHUNK

_BLK = 2048
_GRID = B // _BLK


def _sc_gather_body(u_idx_hbm, i_idx_hbm, eu_hbm, ei_hbm, ubt_hbm, ibt_hbm,
                    u_out, i_out, ub_out, ib_out,
                    uidx_v, iidx_v, gu, gi, ubv, ibv, sem, semb):
    wid = lax.axis_index("s") * _NC + lax.axis_index("c")
    base = wid * _BPW
    pltpu.sync_copy(u_idx_hbm.at[pl.ds(base, _BPW)], uidx_v)
    pltpu.sync_copy(i_idx_hbm.at[pl.ds(base, _BPW)], iidx_v)

    for c in range(_CPW):
        co = c * _CHUNK
        pltpu.async_copy(ubt_hbm.at[uidx_v.at[pl.ds(co, _CHUNK)]],
                         ubv.at[pl.ds(co, _CHUNK)], semb)
        pltpu.async_copy(ibt_hbm.at[iidx_v.at[pl.ds(co, _CHUNK)]],
                         ibv.at[pl.ds(co, _CHUNK)], semb)

    def fbody(f, _):
        fo = f * _BPW
        for c in range(_CPW):
            co = c * _CHUNK
            pltpu.async_copy(
                eu_hbm.at[f].at[uidx_v.at[pl.ds(co, _CHUNK)]],
                gu.at[pl.ds(fo + co, _CHUNK)], sem)
            pltpu.async_copy(
                ei_hbm.at[f].at[iidx_v.at[pl.ds(co, _CHUNK)]],
                gi.at[pl.ds(fo + co, _CHUNK)], sem)
        return ()

    lax.fori_loop(0, F, fbody, (), unroll=False)

    pltpu.make_async_copy(ubt_hbm.at[pl.ds(0, F * _BPW)], gu, sem).wait()
    pltpu.make_async_copy(ubt_hbm.at[pl.ds(0, F * _BPW)], gi, sem).wait()
    pltpu.make_async_copy(ubt_hbm.at[pl.ds(0, _BPW)], ubv, semb).wait()
    pltpu.make_async_copy(ubt_hbm.at[pl.ds(0, _BPW)], ibv, semb).wait()

    for f in range(F):
        pltpu.sync_copy(gu.at[pl.ds(f * _BPW, _BPW)],
                        u_out.at[pl.ds(f * B + base, _BPW)])
        pltpu.sync_copy(gi.at[pl.ds(f * _BPW, _BPW)],
                        i_out.at[pl.ds(f * B + base, _BPW)])
    pltpu.sync_copy(ubv, ub_out.at[pl.ds(base, _BPW)])
    pltpu.sync_copy(ibv, ib_out.at[pl.ds(base, _BPW)])


_sc_gather = pl.kernel(
    _sc_gather_body,
    out_type=[
        jax.ShapeDtypeStruct((F * B,), jnp.float32),
        jax.ShapeDtypeStruct((F * B,), jnp.float32),
        jax.ShapeDtypeStruct((B,), jnp.float32),
        jax.ShapeDtypeStruct((B,), jnp.float32),
    ],
    mesh=plsc.VectorSubcoreMesh(core_axis_name="c", subcore_axis_name="s"),
    scratch_types=[
        pltpu.VMEM((_BPW,), jnp.int32),
        pltpu.VMEM((_BPW,), jnp.int32),
        pltpu.VMEM((F * _BPW,), jnp.float32),
        pltpu.VMEM((F * _BPW,), jnp.float32),
        pltpu.VMEM((_BPW,), jnp.float32),
        pltpu.VMEM((_BPW,), jnp.float32),
        pltpu.SemaphoreType.DMA,
        pltpu.SemaphoreType.DMA,
    ],
    compiler_params=pltpu.CompilerParams(use_tc_tiling_on_sc=False),
)


def _leaky(x):
    return jnp.where(x >= 0, x, 0.1 * x)


def _dense_body(avg_ref, u_ref, i_ref, ub_ref, ib_ref, r_ref,
                w1_ref, b1_ref, w2_ref, b2_ref,
                loss_ref, loss2_ref, acc_ref):
    g = pl.program_id(0)

    @pl.when(g == 0)
    def _init():
        acc_ref[0] = 0.0
        acc_ref[1] = 0.0
        acc_ref[2] = 0.0

    w1 = w1_ref[...]
    w2 = w2_ref[...]
    b1 = b1_ref[...]
    b2 = b2_ref[...]
    cdims = (((0,), (0,)), ((), ()))
    hu = _leaky(lax.dot_general(w1, u_ref[...], cdims,
                                preferred_element_type=jnp.float32) + b1)
    uo = _leaky(lax.dot_general(w2, hu, cdims,
                                preferred_element_type=jnp.float32) + b2)
    hi = _leaky(lax.dot_general(w1, i_ref[...], cdims,
                                preferred_element_type=jnp.float32) + b1)
    io = _leaky(lax.dot_general(w2, hi, cdims,
                                preferred_element_type=jnp.float32) + b2)
    pred = (jnp.sum(uo * io, axis=0, keepdims=True)
            + ub_ref[...] + ib_ref[...] + avg_ref[0])
    diff = pred - r_ref[...]
    acc_ref[0] += jnp.sum(diff * diff)
    acc_ref[1] += jnp.sum(uo * uo)
    acc_ref[2] += jnp.sum(io * io)

    @pl.when(g == pl.num_programs(0) - 1)
    def _fin():
        loss2 = acc_ref[0] / B
        l2 = LAMBDA * (acc_ref[1] + acc_ref[2]) / (B * F)
        loss2_ref[0, 0] = loss2
        loss_ref[0, 0] = loss2 + l2


def _dense(avg, u, it, ub, ib, r, w1, b1, w2, b2, interpret=False):
    return pl.pallas_call(
        _dense_body,
        grid=(_GRID,),
        in_specs=[
            pl.BlockSpec(memory_space=pltpu.SMEM),
            pl.BlockSpec((F, _BLK), lambda i: (0, i)),
            pl.BlockSpec((F, _BLK), lambda i: (0, i)),
            pl.BlockSpec((1, _BLK), lambda i: (0, i)),
            pl.BlockSpec((1, _BLK), lambda i: (0, i)),
            pl.BlockSpec((1, _BLK), lambda i: (0, i)),
            pl.BlockSpec((F, H), lambda i: (0, 0)),
            pl.BlockSpec((H, 1), lambda i: (0, 0)),
            pl.BlockSpec((H, F), lambda i: (0, 0)),
            pl.BlockSpec((F, 1), lambda i: (0, 0)),
        ],
        out_specs=[
            pl.BlockSpec(memory_space=pltpu.SMEM),
            pl.BlockSpec(memory_space=pltpu.SMEM),
        ],
        out_shape=[
            jax.ShapeDtypeStruct((1, 1), jnp.float32),
            jax.ShapeDtypeStruct((1, 1), jnp.float32),
        ],
        scratch_shapes=[pltpu.SMEM((3,), jnp.float32)],
        interpret=interpret,
    )(avg, u, it, ub, ib, r, w1, b1, w2, b2)


def kernel(user0, item_i0, ratings, embed_user, embed_item,
           user_bias_tab, item_bias_tab, W1, b1, W2, b2, avg_rating):
    u_idx = user0.astype(jnp.int32)
    i_idx = item_i0.astype(jnp.int32)
    uT_flat, iT_flat, ub_g, ib_g = _sc_gather(
        u_idx, i_idx, embed_user.T, embed_item.T,
        user_bias_tab.reshape(-1), item_bias_tab.reshape(-1))
    loss, loss2 = _dense(
        avg_rating,
        uT_flat.reshape(F, B), iT_flat.reshape(F, B),
        ub_g.reshape(1, B), ib_g.reshape(1, B),
        ratings.astype(jnp.float32).reshape(1, B),
        W1, b1.reshape(H, 1), W2, b2.reshape(F, 1))
    return (loss[0, 0], loss2[0, 0], 0.0, 0.0)


# speedup vs baseline: 3.3328x; 3.3328x over previous
import jax
import jax.numpy as jnp
from jax import lax
from jax.experimental import pallas as pl
from jax.experimental.pallas import tpu as pltpu
from jax.experimental.pallas import tpu_sc as plsc

B = 16384
F = 32
H = 64
LAMBDA = 0.001

_NC = 2
_NS = 16
_NW = _NC * _NS
_CHUNK = 128
_ROWS = B // _CHUNK
_CPW = _ROWS // _NW

_BLK = 2048
_GRID = B // _BLK


def _sc_gather_body(u_idx_hbm, i_idx_hbm, eu_hbm, ei_hbm, ubt_hbm, ibt_hbm,
                    u_out, i_out, ub_out, ib_out,
                    uidx_v, iidx_v, urows_v, irows_v, ubv, ibv, sem):
    wid = lax.axis_index("s") * _NC + lax.axis_index("c")
    base = wid * _CPW
    pltpu.sync_copy(u_idx_hbm.at[pl.ds(base, _CPW)], uidx_v)
    pltpu.sync_copy(i_idx_hbm.at[pl.ds(base, _CPW)], iidx_v)
    copies = []
    for j in range(_CPW):
        copies.append(pltpu.async_copy(eu_hbm.at[uidx_v.at[j]], urows_v.at[j], sem))
        copies.append(pltpu.async_copy(ei_hbm.at[iidx_v.at[j]], irows_v.at[j], sem))
        copies.append(pltpu.async_copy(ubt_hbm.at[uidx_v.at[j]], ubv.at[j], sem))
        copies.append(pltpu.async_copy(ibt_hbm.at[iidx_v.at[j]], ibv.at[j], sem))
    for c in copies:
        c.wait()
    pltpu.sync_copy(urows_v, u_out.at[pl.ds(base, _CPW)])
    pltpu.sync_copy(irows_v, i_out.at[pl.ds(base, _CPW)])
    pltpu.sync_copy(ubv, ub_out.at[pl.ds(base, _CPW)])
    pltpu.sync_copy(ibv, ib_out.at[pl.ds(base, _CPW)])


_sc_gather = pl.kernel(
    _sc_gather_body,
    out_type=[
        jax.ShapeDtypeStruct((_ROWS, _CHUNK, F), jnp.float32),
        jax.ShapeDtypeStruct((_ROWS, _CHUNK, F), jnp.float32),
        jax.ShapeDtypeStruct((_ROWS, _CHUNK), jnp.float32),
        jax.ShapeDtypeStruct((_ROWS, _CHUNK), jnp.float32),
    ],
    mesh=plsc.VectorSubcoreMesh(core_axis_name="c", subcore_axis_name="s"),
    scratch_types=[
        pltpu.VMEM((_CPW, _CHUNK), jnp.int32),
        pltpu.VMEM((_CPW, _CHUNK), jnp.int32),
        pltpu.VMEM((_CPW, _CHUNK, F), jnp.float32),
        pltpu.VMEM((_CPW, _CHUNK, F), jnp.float32),
        pltpu.VMEM((_CPW, _CHUNK), jnp.float32),
        pltpu.VMEM((_CPW, _CHUNK), jnp.float32),
        pltpu.SemaphoreType.DMA,
    ],
    compiler_params=pltpu.CompilerParams(use_tc_tiling_on_sc=False),
)


def _leaky(x):
    return jnp.where(x >= 0, x, 0.1 * x)


def _dense_body(avg_ref, u_ref, i_ref, ub_ref, ib_ref, r_ref,
                w1_ref, b1_ref, w2_ref, b2_ref,
                loss_ref, loss2_ref, acc_ref):
    g = pl.program_id(0)

    @pl.when(g == 0)
    def _init():
        acc_ref[0] = 0.0
        acc_ref[1] = 0.0
        acc_ref[2] = 0.0

    w1 = w1_ref[...]
    w2 = w2_ref[...]
    b1 = b1_ref[...]
    b2 = b2_ref[...]
    hu = _leaky(jnp.dot(u_ref[...], w1, preferred_element_type=jnp.float32) + b1)
    uo = _leaky(jnp.dot(hu, w2, preferred_element_type=jnp.float32) + b2)
    hi = _leaky(jnp.dot(i_ref[...], w1, preferred_element_type=jnp.float32) + b1)
    io = _leaky(jnp.dot(hi, w2, preferred_element_type=jnp.float32) + b2)
    pred = (jnp.sum(uo * io, axis=1, keepdims=True)
            + ub_ref[...] + ib_ref[...] + avg_ref[0])
    diff = pred - r_ref[...]
    acc_ref[0] += jnp.sum(diff * diff)
    acc_ref[1] += jnp.sum(uo * uo)
    acc_ref[2] += jnp.sum(io * io)

    @pl.when(g == pl.num_programs(0) - 1)
    def _fin():
        loss2 = acc_ref[0] / B
        l2 = LAMBDA * (acc_ref[1] + acc_ref[2]) / (B * F)
        loss2_ref[0, 0] = loss2
        loss_ref[0, 0] = loss2 + l2


def _dense(avg, u, it, ub, ib, r, w1, b1, w2, b2, interpret=False):
    return pl.pallas_call(
        _dense_body,
        grid=(_GRID,),
        in_specs=[
            pl.BlockSpec(memory_space=pltpu.SMEM),
            pl.BlockSpec((_BLK, F), lambda i: (i, 0)),
            pl.BlockSpec((_BLK, F), lambda i: (i, 0)),
            pl.BlockSpec((_BLK, 1), lambda i: (i, 0)),
            pl.BlockSpec((_BLK, 1), lambda i: (i, 0)),
            pl.BlockSpec((_BLK, 1), lambda i: (i, 0)),
            pl.BlockSpec((F, H), lambda i: (0, 0)),
            pl.BlockSpec((1, H), lambda i: (0, 0)),
            pl.BlockSpec((H, F), lambda i: (0, 0)),
            pl.BlockSpec((1, F), lambda i: (0, 0)),
        ],
        out_specs=[
            pl.BlockSpec(memory_space=pltpu.SMEM),
            pl.BlockSpec(memory_space=pltpu.SMEM),
        ],
        out_shape=[
            jax.ShapeDtypeStruct((1, 1), jnp.float32),
            jax.ShapeDtypeStruct((1, 1), jnp.float32),
        ],
        scratch_shapes=[pltpu.SMEM((3,), jnp.float32)],
        interpret=interpret,
    )(avg, u, it, ub, ib, r, w1, b1, w2, b2)


def kernel(user0, item_i0, ratings, embed_user, embed_item,
           user_bias_tab, item_bias_tab, W1, b1, W2, b2, avg_rating):
    u_idx = user0.astype(jnp.int32).reshape(_ROWS, _CHUNK)
    i_idx = item_i0.astype(jnp.int32).reshape(_ROWS, _CHUNK)
    u_g, i_g, ub_g, ib_g = _sc_gather(
        u_idx, i_idx, embed_user, embed_item,
        user_bias_tab.reshape(-1), item_bias_tab.reshape(-1))
    loss, loss2 = _dense(
        avg_rating,
        u_g.reshape(B, F), i_g.reshape(B, F),
        ub_g.reshape(B, 1), ib_g.reshape(B, 1),
        ratings.astype(jnp.float32).reshape(B, 1),
        W1, b1.reshape(1, H), W2, b2.reshape(1, F))
    return (loss[0, 0], loss2[0, 0], 0.0, 0.0)
